# baseline (speedup 1.0000x reference)
import jax
import jax.numpy as jnp
from jax import lax
from jax.experimental import pallas as pl
from jax.experimental.pallas import tpu as pltpu
from jax.experimental.pallas import tpu_sc as plsc

N = 10000
E = 160000
EP = 163840
IN = 128
K = 10
H = 40
OUT = 40
KP = 16
HP = 48
ROW = K * HP
ROWP = 512
NC = 2
NS = 16
NW = NC * NS
EPW = EP // NW
B = 32
NB = EPW // B
NP = 10240
FW = NP * HP
ZW = FW // NS
BW = B * HP


def _mm_body(x_ref, w_ref, o_ref):
    o_ref[...] = jnp.dot(x_ref[...], w_ref[...],
                         preferred_element_type=jnp.float32)


def _matmul(x, w, bm):
    m, kk = x.shape
    _, n = w.shape
    return pl.pallas_call(
        _mm_body,
        grid=(m // bm,),
        in_specs=[pl.BlockSpec((bm, kk), lambda i: (i, 0)),
                  pl.BlockSpec((kk, n), lambda i: (0, 0))],
        out_specs=pl.BlockSpec((bm, n), lambda i: (i, 0)),
        out_shape=jax.ShapeDtypeStruct((m, n), jnp.float32),
    )(x, w)


def _g_body(p_ref, m_ref, o_ref):
    o_ref[...] = jnp.exp(jnp.dot(p_ref[...], m_ref[...],
                                 preferred_element_type=jnp.float32))


def _gmm_weights(P, M, be=4000):
    return pl.pallas_call(
        _g_body,
        grid=(E // be,),
        in_specs=[pl.BlockSpec((be, 8), lambda i: (i, 0)),
                  pl.BlockSpec((8, KP), lambda i: (0, 0))],
        out_specs=pl.BlockSpec((be, KP), lambda i: (i, 0)),
        out_shape=jax.ShapeDtypeStruct((E, KP), jnp.float32),
    )(P, M)


_sc_mesh = plsc.VectorSubcoreMesh(core_axis_name="c", subcore_axis_name="s",
                                  num_cores=NC, num_subcores=NS)


_SC_SCRATCH = [
    pltpu.VMEM((B,), jnp.int32), pltpu.VMEM((B,), jnp.int32),
    pltpu.VMEM((B,), jnp.int32), pltpu.VMEM((B,), jnp.int32),
    pltpu.VMEM((B, KP), jnp.float32), pltpu.VMEM((B, KP), jnp.float32),
    pltpu.VMEM((B, ROWP), jnp.float32), pltpu.VMEM((B, ROWP), jnp.float32),
    pltpu.VMEM((BW,), jnp.float32), pltpu.VMEM((BW,), jnp.float32),
    pltpu.VMEM((BW,), jnp.int32), pltpu.VMEM((BW,), jnp.int32),
    pltpu.VMEM_SHARED((FW,), jnp.float32),
    pltpu.SemaphoreType.DMA,
    pltpu.SemaphoreType.DMA,
    pltpu.SemaphoreType.DMA,
]


def _sc_body(h_hbm, src_hbm, dst_hbm, g_hbm, out_hbm,
             src0, src1, dst0, dst1, g0, g1, rows0, rows1,
             msg0, msg1, idx0, idx1,
             acc_sh, msem, gsem, ssem):
    c = lax.axis_index("c")
    s = lax.axis_index("s")
    wid = s * NC + c
    base = wid * EPW

    srcs, dsts, gs = [src0, src1], [dst0, dst1], [g0, g1]
    rows, msgs, idxs = [rows0, rows1], [msg0, msg1], [idx0, idx1]

    zeros16 = jnp.zeros((16,), jnp.float32)
    iota16 = lax.iota(jnp.int32, 16)

    def _zrow(i, carry):
        msg0[pl.ds(i * 16, 16)] = zeros16
        return carry

    lax.fori_loop(0, BW // 16, _zrow, 0)

    def _zblk(j, carry):
        pltpu.sync_copy(msg0, acc_sh.at[pl.ds(s * ZW + j * BW, BW)])
        return carry

    lax.fori_loop(0, ZW // BW, _zblk, 0)
    plsc.subcore_barrier()

    deg_marker = jnp.where(iota16 == 8, jnp.float32(1.0), jnp.float32(0.0))

    def meta_issue(blk, mb):
        e0 = base + blk * B
        pltpu.async_copy(src_hbm.at[pl.ds(e0, B)], srcs[mb], msem)
        pltpu.async_copy(dst_hbm.at[pl.ds(e0, B)], dsts[mb], msem)
        pltpu.async_copy(g_hbm.at[pl.ds(e0, B)], gs[mb], msem)

    def meta_wait(mb):
        pltpu.make_async_copy(src_hbm.at[pl.ds(0, B)], srcs[mb], msem).wait()
        pltpu.make_async_copy(dst_hbm.at[pl.ds(0, B)], dsts[mb], msem).wait()
        pltpu.make_async_copy(g_hbm.at[pl.ds(0, B)], gs[mb], msem).wait()

    def gather_issue(mb, rb):
        pltpu.async_copy(h_hbm.at[srcs[mb]], rows[rb], gsem)

    def gather_wait(rb):
        pltpu.make_async_copy(h_hbm.at[srcs[0]], rows[rb], gsem).wait()

    def scat_issue(pb):
        pltpu.async_copy(msgs[pb], acc_sh.at[idxs[pb]], ssem, add=True)

    def scat_wait(pb):
        pltpu.make_async_copy(msgs[pb], acc_sh.at[idxs[pb]], ssem).wait()

    def _compute(bi):
        dst_v, g_v, rows_v, msg_v, idx_v = (
            dsts[bi], gs[bi], rows[bi], msgs[bi], idxs[bi])
        for q in range(B // 16):
            dgrp = dst_v[pl.ds(q * 16, 16)]
            for l in range(16):
                dm = lax.broadcast(dgrp[l] * HP, (16,))
                mrow = (q * 16 + l) * HP
                for j in range(HP // 16):
                    idx_v[pl.ds(mrow + j * 16, 16)] = dm + (iota16 + j * 16)

        def _tree(ts):
            while len(ts) > 1:
                nxt = [ts[i] + ts[i + 1] for i in range(0, len(ts) - 1, 2)]
                if len(ts) % 2:
                    nxt.append(ts[-1])
                ts = nxt
            return ts[0]

        @plsc.parallel_loop(0, B, step=1, unroll=2)
        def _edge(b):
            g_row = g_v[b]
            gks = [lax.broadcast(g_row[k], (16,)) for k in range(K)]
            t0 = [rows_v[b, pl.ds(k * HP, 16)] * gks[k] for k in range(K)]
            t1 = [rows_v[b, pl.ds(k * HP + 16, 16)] * gks[k] for k in range(K)]
            t2 = [rows_v[b, pl.ds(k * HP + 32, 16)] * gks[k] for k in range(K)]
            msg_v[pl.ds(b * HP, 16)] = _tree(t0)
            msg_v[pl.ds(b * HP + 16, 16)] = _tree(t1)
            msg_v[pl.ds(b * HP + 32, 16)] = _tree(t2 + [deg_marker])

    def _sub(i, bi):
        @pl.when(i + 1 < NB)
        def _():
            meta_wait(1 - bi)
            gather_issue(1 - bi, 1 - bi)
        gather_wait(bi)
        @pl.when(i >= 2)
        def _():
            scat_wait(bi)
        _compute(bi)
        @pl.when(i + 2 < NB)
        def _():
            meta_issue(i + 2, bi)
        scat_issue(bi)

    meta_issue(0, 0)
    meta_wait(0)
    gather_issue(0, 0)
    meta_issue(1, 1)

    def _pair(i2, carry):
        i = i2 * 2
        _sub(i, 0)
        _sub(i + 1, 1)
        return carry

    lax.fori_loop(0, NB // 2, _pair, 0)

    scat_wait(0)
    scat_wait(1)
    plsc.subcore_barrier()

    def _wblk(j, carry):
        o0 = s * ZW + j * BW
        pltpu.sync_copy(acc_sh.at[pl.ds(o0, BW)], msg0)
        pltpu.sync_copy(msg0, out_hbm.at[c, pl.ds(o0, BW)])
        return carry

    lax.fori_loop(0, ZW // BW, _wblk, 0)


_sc_aggregate_impl = pl.kernel(
    _sc_body,
    out_type=jax.ShapeDtypeStruct((NC, FW), jnp.float32),
    mesh=_sc_mesh,
    scratch_types=_SC_SCRATCH,
)


def _sc_aggregate(h, src, dst, g):
    acc = _sc_aggregate_impl(h, src, dst, g)
    return acc.reshape(NC, NP, HP)[:, :N, :]


def _layer2_body(a0_ref, a1_ref, b_ref, w_ref, o_ref):
    s = a0_ref[...] + a1_ref[...]
    col = lax.broadcasted_iota(jnp.int32, s.shape, 1)
    deg = jnp.max(jnp.where(col == H, s, -jnp.inf), axis=1, keepdims=True)
    deg = jnp.maximum(deg, 1.0)
    hfeat = jnp.maximum(s / deg + b_ref[...], 0.0)
    o_ref[...] = jnp.dot(hfeat, w_ref[...],
                         preferred_element_type=jnp.float32)


def _final_body(a0_ref, a1_ref, b_ref, o_ref):
    s = a0_ref[...] + a1_ref[...]
    col = lax.broadcasted_iota(jnp.int32, s.shape, 1)
    deg = jnp.max(jnp.where(col == OUT, s, -jnp.inf), axis=1, keepdims=True)
    deg = jnp.maximum(deg, 1.0)
    o_ref[...] = jnp.tanh(s / deg + b_ref[...])


def kernel(n_feat, edge_index, p_coord, W1, mu1, inv_sigma1, b1,
           W2, mu2, inv_sigma2, b2):
    f32 = jnp.float32
    src = edge_index[0].astype(jnp.int32)
    dst = edge_index[1].astype(jnp.int32)
    p = p_coord.astype(f32)

    def padw(W, in_dim, out_dim):
        Wk = W.reshape(in_dim, K, out_dim)
        Wk = jnp.pad(Wk, ((0, 0), (0, 0), (0, HP - out_dim)))
        Wk = Wk.reshape(in_dim, K * HP)
        return jnp.pad(Wk, ((0, 0), (0, ROWP - ROW)))

    W1p = padw(W1, IN, H)
    W2p = jnp.pad(padw(W2, H, OUT), ((0, HP - H), (0, 0)))
    b1p = jnp.pad(b1, (0, HP - H)).reshape(1, HP)
    b2p = jnp.pad(b2, (0, HP - OUT)).reshape(1, HP)

    def build_M(mu, inv):
        iv2 = (inv * inv).astype(f32)
        rows = jnp.concatenate([
            (-0.5 * iv2).T,
            (iv2 * mu).T,
            (-0.5 * (iv2 * mu * mu).sum(axis=1))[None, :],
            jnp.zeros((3, K), f32),
        ], axis=0)
        return jnp.pad(rows, ((0, 0), (0, KP - K)))

    M1 = build_M(mu1, inv_sigma1)
    M2 = build_M(mu2, inv_sigma2)
    P = jnp.concatenate(
        [p * p, p, jnp.ones((E, 1), f32), jnp.zeros((E, 3), f32)], axis=1)

    g1 = _gmm_weights(P, M1)
    g2 = _gmm_weights(P, M2)
    h1p = _matmul(n_feat, W1p, 1000)

    pad = EP - E
    src_p = jnp.concatenate([src, jnp.zeros((pad,), jnp.int32)])
    dst_p = jnp.concatenate([dst, jnp.full((pad,), N, jnp.int32)])
    g1p = jnp.concatenate([g1, jnp.zeros((pad, KP), f32)])
    g2p = jnp.concatenate([g2, jnp.zeros((pad, KP), f32)])

    acc1 = _sc_aggregate(h1p, src_p, dst_p, g1p)

    bm = 1000
    h2p = pl.pallas_call(
        _layer2_body,
        grid=(N // bm,),
        in_specs=[pl.BlockSpec((bm, HP), lambda i: (i, 0)),
                  pl.BlockSpec((bm, HP), lambda i: (i, 0)),
                  pl.BlockSpec((1, HP), lambda i: (0, 0)),
                  pl.BlockSpec((HP, ROWP), lambda i: (0, 0))],
        out_specs=pl.BlockSpec((bm, ROWP), lambda i: (i, 0)),
        out_shape=jax.ShapeDtypeStruct((N, ROWP), f32),
    )(acc1[0], acc1[1], b1p, W2p)

    acc2 = _sc_aggregate(h2p, src_p, dst_p, g2p)

    out48 = pl.pallas_call(
        _final_body,
        grid=(N // bm,),
        in_specs=[pl.BlockSpec((bm, HP), lambda i: (i, 0)),
                  pl.BlockSpec((bm, HP), lambda i: (i, 0)),
                  pl.BlockSpec((1, HP), lambda i: (0, 0))],
        out_specs=pl.BlockSpec((bm, HP), lambda i: (i, 0)),
        out_shape=jax.ShapeDtypeStruct((N, HP), f32),
    )(acc2[0], acc2[1], b2p)

    return out48[:, :OUT]


# speedup vs baseline: 1.0000x; 1.0000x over previous
import jax
import jax.numpy as jnp
from jax import lax
from jax.experimental import pallas as pl
from jax.experimental.pallas import tpu as pltpu
from jax.experimental.pallas import tpu_sc as plsc

N = 10000
E = 160000
EP = 163840
IN = 128
K = 10
H = 40
OUT = 40
KP = 16
HP = 48
ROW = K * HP
ROWP = 512
NC = 2
NS = 16
NW = NC * NS
EPW = EP // NW
B = 32
NB = EPW // B
NP = 10240
FW = NP * HP
ZW = FW // NS
BW = B * HP


def _mm_body(x_ref, w_ref, o_ref):
    o_ref[...] = jnp.dot(x_ref[...], w_ref[...],
                         preferred_element_type=jnp.float32)


def _matmul(x, w, bm):
    m, kk = x.shape
    _, n = w.shape
    return pl.pallas_call(
        _mm_body,
        grid=(m // bm,),
        in_specs=[pl.BlockSpec((bm, kk), lambda i: (i, 0)),
                  pl.BlockSpec((kk, n), lambda i: (0, 0))],
        out_specs=pl.BlockSpec((bm, n), lambda i: (i, 0)),
        out_shape=jax.ShapeDtypeStruct((m, n), jnp.float32),
    )(x, w)


def _g_body(p_ref, m_ref, o_ref):
    o_ref[...] = jnp.exp(jnp.dot(p_ref[...], m_ref[...],
                                 preferred_element_type=jnp.float32))


def _gmm_weights(P, M, be=4000):
    return pl.pallas_call(
        _g_body,
        grid=(E // be,),
        in_specs=[pl.BlockSpec((be, 8), lambda i: (i, 0)),
                  pl.BlockSpec((8, KP), lambda i: (0, 0))],
        out_specs=pl.BlockSpec((be, KP), lambda i: (i, 0)),
        out_shape=jax.ShapeDtypeStruct((E, KP), jnp.float32),
    )(P, M)


_sc_mesh = plsc.VectorSubcoreMesh(core_axis_name="c", subcore_axis_name="s",
                                  num_cores=NC, num_subcores=NS)


_SC_SCRATCH = [
    pltpu.VMEM((B,), jnp.int32), pltpu.VMEM((B,), jnp.int32),
    pltpu.VMEM((B,), jnp.int32), pltpu.VMEM((B,), jnp.int32),
    pltpu.VMEM((B, KP), jnp.float32), pltpu.VMEM((B, KP), jnp.float32),
    pltpu.VMEM((B, ROWP), jnp.float32), pltpu.VMEM((B, ROWP), jnp.float32),
    pltpu.VMEM((BW,), jnp.float32), pltpu.VMEM((BW,), jnp.float32),
    pltpu.VMEM((BW,), jnp.int32), pltpu.VMEM((BW,), jnp.int32),
    pltpu.VMEM_SHARED((FW,), jnp.float32),
    pltpu.SemaphoreType.DMA,
    pltpu.SemaphoreType.DMA,
    pltpu.SemaphoreType.DMA,
]


def _sc_body(h_hbm, src_hbm, dst_hbm, g_hbm, out_hbm,
             src0, src1, dst0, dst1, g0, g1, rows0, rows1,
             msg0, msg1, idx0, idx1,
             acc_sh, msem, gsem, ssem):
    c = lax.axis_index("c")
    s = lax.axis_index("s")
    wid = s * NC + c
    base = wid * EPW

    srcs, dsts, gs = [src0, src1], [dst0, dst1], [g0, g1]
    rows, msgs, idxs = [rows0, rows1], [msg0, msg1], [idx0, idx1]

    zeros16 = jnp.zeros((16,), jnp.float32)
    iota16 = lax.iota(jnp.int32, 16)

    def _zrow(i, carry):
        msg0[pl.ds(i * 16, 16)] = zeros16
        return carry

    lax.fori_loop(0, BW // 16, _zrow, 0)

    def _zblk(j, carry):
        pltpu.sync_copy(msg0, acc_sh.at[pl.ds(s * ZW + j * BW, BW)])
        return carry

    lax.fori_loop(0, ZW // BW, _zblk, 0)
    plsc.subcore_barrier()

    deg_marker = jnp.where(iota16 == 8, jnp.float32(1.0), jnp.float32(0.0))

    def meta_issue(blk, mb):
        e0 = base + blk * B
        pltpu.async_copy(src_hbm.at[pl.ds(e0, B)], srcs[mb], msem)
        pltpu.async_copy(dst_hbm.at[pl.ds(e0, B)], dsts[mb], msem)
        pltpu.async_copy(g_hbm.at[pl.ds(e0, B)], gs[mb], msem)

    def meta_wait(mb):
        pltpu.make_async_copy(src_hbm.at[pl.ds(0, B)], srcs[mb], msem).wait()
        pltpu.make_async_copy(dst_hbm.at[pl.ds(0, B)], dsts[mb], msem).wait()
        pltpu.make_async_copy(g_hbm.at[pl.ds(0, B)], gs[mb], msem).wait()

    def gather_issue(mb, rb):
        h = B // 2
        pltpu.async_copy(h_hbm.at[srcs[mb].at[pl.ds(0, h)]],
                         rows[rb].at[pl.ds(0, h)], gsem)
        pltpu.async_copy(h_hbm.at[srcs[mb].at[pl.ds(h, h)]],
                         rows[rb].at[pl.ds(h, h)], gsem)

    def gather_wait(rb):
        h = B // 2
        pltpu.make_async_copy(h_hbm.at[srcs[0].at[pl.ds(0, h)]],
                              rows[rb].at[pl.ds(0, h)], gsem).wait()
        pltpu.make_async_copy(h_hbm.at[srcs[0].at[pl.ds(h, h)]],
                              rows[rb].at[pl.ds(h, h)], gsem).wait()

    def scat_issue(pb):
        pltpu.async_copy(msgs[pb], acc_sh.at[idxs[pb]], ssem, add=True)

    def scat_wait(pb):
        pltpu.make_async_copy(msgs[pb], acc_sh.at[idxs[pb]], ssem).wait()

    def _compute(bi):
        dst_v, g_v, rows_v, msg_v, idx_v = (
            dsts[bi], gs[bi], rows[bi], msgs[bi], idxs[bi])
        for q in range(B // 16):
            dgrp = dst_v[pl.ds(q * 16, 16)]
            for l in range(16):
                dm = lax.broadcast(dgrp[l] * HP, (16,))
                mrow = (q * 16 + l) * HP
                for j in range(HP // 16):
                    idx_v[pl.ds(mrow + j * 16, 16)] = dm + (iota16 + j * 16)

        def _tree(ts):
            while len(ts) > 1:
                nxt = [ts[i] + ts[i + 1] for i in range(0, len(ts) - 1, 2)]
                if len(ts) % 2:
                    nxt.append(ts[-1])
                ts = nxt
            return ts[0]

        @plsc.parallel_loop(0, B, step=1, unroll=2)
        def _edge(b):
            g_row = g_v[b]
            gks = [lax.broadcast(g_row[k], (16,)) for k in range(K)]
            t0 = [rows_v[b, pl.ds(k * HP, 16)] * gks[k] for k in range(K)]
            t1 = [rows_v[b, pl.ds(k * HP + 16, 16)] * gks[k] for k in range(K)]
            t2 = [rows_v[b, pl.ds(k * HP + 32, 16)] * gks[k] for k in range(K)]
            msg_v[pl.ds(b * HP, 16)] = _tree(t0)
            msg_v[pl.ds(b * HP + 16, 16)] = _tree(t1)
            msg_v[pl.ds(b * HP + 32, 16)] = _tree(t2 + [deg_marker])

    def _sub(i, bi):
        @pl.when(i + 1 < NB)
        def _():
            meta_wait(1 - bi)
            gather_issue(1 - bi, 1 - bi)
        gather_wait(bi)
        @pl.when(i >= 2)
        def _():
            scat_wait(bi)
        _compute(bi)
        @pl.when(i + 2 < NB)
        def _():
            meta_issue(i + 2, bi)
        scat_issue(bi)

    meta_issue(0, 0)
    meta_wait(0)
    gather_issue(0, 0)
    meta_issue(1, 1)

    def _pair(i2, carry):
        i = i2 * 2
        _sub(i, 0)
        _sub(i + 1, 1)
        return carry

    lax.fori_loop(0, NB // 2, _pair, 0)

    scat_wait(0)
    scat_wait(1)
    plsc.subcore_barrier()

    def _wblk(j, carry):
        o0 = s * ZW + j * BW
        pltpu.sync_copy(acc_sh.at[pl.ds(o0, BW)], msg0)
        pltpu.sync_copy(msg0, out_hbm.at[c, pl.ds(o0, BW)])
        return carry

    lax.fori_loop(0, ZW // BW, _wblk, 0)


_sc_aggregate_impl = pl.kernel(
    _sc_body,
    out_type=jax.ShapeDtypeStruct((NC, FW), jnp.float32),
    mesh=_sc_mesh,
    scratch_types=_SC_SCRATCH,
)


def _sc_aggregate(h, src, dst, g):
    acc = _sc_aggregate_impl(h, src, dst, g)
    return acc.reshape(NC, NP, HP)[:, :N, :]


def _layer2_body(a0_ref, a1_ref, b_ref, w_ref, o_ref):
    s = a0_ref[...] + a1_ref[...]
    col = lax.broadcasted_iota(jnp.int32, s.shape, 1)
    deg = jnp.max(jnp.where(col == H, s, -jnp.inf), axis=1, keepdims=True)
    deg = jnp.maximum(deg, 1.0)
    hfeat = jnp.maximum(s / deg + b_ref[...], 0.0)
    o_ref[...] = jnp.dot(hfeat, w_ref[...],
                         preferred_element_type=jnp.float32)


def _final_body(a0_ref, a1_ref, b_ref, o_ref):
    s = a0_ref[...] + a1_ref[...]
    col = lax.broadcasted_iota(jnp.int32, s.shape, 1)
    deg = jnp.max(jnp.where(col == OUT, s, -jnp.inf), axis=1, keepdims=True)
    deg = jnp.maximum(deg, 1.0)
    o_ref[...] = jnp.tanh(s / deg + b_ref[...])


def kernel(n_feat, edge_index, p_coord, W1, mu1, inv_sigma1, b1,
           W2, mu2, inv_sigma2, b2):
    f32 = jnp.float32
    src = edge_index[0].astype(jnp.int32)
    dst = edge_index[1].astype(jnp.int32)
    p = p_coord.astype(f32)

    def padw(W, in_dim, out_dim):
        Wk = W.reshape(in_dim, K, out_dim)
        Wk = jnp.pad(Wk, ((0, 0), (0, 0), (0, HP - out_dim)))
        Wk = Wk.reshape(in_dim, K * HP)
        return jnp.pad(Wk, ((0, 0), (0, ROWP - ROW)))

    W1p = padw(W1, IN, H)
    W2p = jnp.pad(padw(W2, H, OUT), ((0, HP - H), (0, 0)))
    b1p = jnp.pad(b1, (0, HP - H)).reshape(1, HP)
    b2p = jnp.pad(b2, (0, HP - OUT)).reshape(1, HP)

    def build_M(mu, inv):
        iv2 = (inv * inv).astype(f32)
        rows = jnp.concatenate([
            (-0.5 * iv2).T,
            (iv2 * mu).T,
            (-0.5 * (iv2 * mu * mu).sum(axis=1))[None, :],
            jnp.zeros((3, K), f32),
        ], axis=0)
        return jnp.pad(rows, ((0, 0), (0, KP - K)))

    M1 = build_M(mu1, inv_sigma1)
    M2 = build_M(mu2, inv_sigma2)
    P = jnp.concatenate(
        [p * p, p, jnp.ones((E, 1), f32), jnp.zeros((E, 3), f32)], axis=1)

    g1 = _gmm_weights(P, M1)
    g2 = _gmm_weights(P, M2)
    h1p = _matmul(n_feat, W1p, 1000)

    pad = EP - E
    src_p = jnp.concatenate([src, jnp.zeros((pad,), jnp.int32)])
    dst_p = jnp.concatenate([dst, jnp.full((pad,), N, jnp.int32)])
    g1p = jnp.concatenate([g1, jnp.zeros((pad, KP), f32)])
    g2p = jnp.concatenate([g2, jnp.zeros((pad, KP), f32)])

    acc1 = _sc_aggregate(h1p, src_p, dst_p, g1p)

    bm = 1000
    h2p = pl.pallas_call(
        _layer2_body,
        grid=(N // bm,),
        in_specs=[pl.BlockSpec((bm, HP), lambda i: (i, 0)),
                  pl.BlockSpec((bm, HP), lambda i: (i, 0)),
                  pl.BlockSpec((1, HP), lambda i: (0, 0)),
                  pl.BlockSpec((HP, ROWP), lambda i: (0, 0))],
        out_specs=pl.BlockSpec((bm, ROWP), lambda i: (i, 0)),
        out_shape=jax.ShapeDtypeStruct((N, ROWP), f32),
    )(acc1[0], acc1[1], b1p, W2p)

    acc2 = _sc_aggregate(h2p, src_p, dst_p, g2p)

    out48 = pl.pallas_call(
        _final_body,
        grid=(N // bm,),
        in_specs=[pl.BlockSpec((bm, HP), lambda i: (i, 0)),
                  pl.BlockSpec((bm, HP), lambda i: (i, 0)),
                  pl.BlockSpec((1, HP), lambda i: (0, 0))],
        out_specs=pl.BlockSpec((bm, HP), lambda i: (i, 0)),
        out_shape=jax.ShapeDtypeStruct((N, HP), f32),
    )(acc2[0], acc2[1], b2p)

    return out48[:, :OUT]


# speedup vs baseline: 1.0078x; 1.0078x over previous
import jax
import jax.numpy as jnp
from jax import lax
from jax.experimental import pallas as pl
from jax.experimental.pallas import tpu as pltpu
from jax.experimental.pallas import tpu_sc as plsc

N = 10000
E = 160000
EP = 163840
IN = 128
K = 10
H = 40
OUT = 40
KP = 16
HP = 48
ROW = K * HP
ROWP = 512
NC = 2
NS = 16
NW = NC * NS
EPW = EP // NW
B = 32
NB = EPW // B
NP = 10240
FW = NP * HP
ZW = FW // NS
BW = B * HP


def _mm_body(x_ref, w_ref, o_ref):
    o_ref[...] = jnp.dot(x_ref[...], w_ref[...],
                         preferred_element_type=jnp.float32)


def _matmul(x, w, bm):
    m, kk = x.shape
    _, n = w.shape
    return pl.pallas_call(
        _mm_body,
        grid=(m // bm,),
        in_specs=[pl.BlockSpec((bm, kk), lambda i: (i, 0)),
                  pl.BlockSpec((kk, n), lambda i: (0, 0))],
        out_specs=pl.BlockSpec((bm, n), lambda i: (i, 0)),
        out_shape=jax.ShapeDtypeStruct((m, n), jnp.float32),
    )(x, w)


def _g_body(p_ref, m_ref, o_ref):
    o_ref[...] = jnp.exp(jnp.dot(p_ref[...], m_ref[...],
                                 preferred_element_type=jnp.float32))


def _gmm_weights(P, M, be=4000):
    return pl.pallas_call(
        _g_body,
        grid=(E // be,),
        in_specs=[pl.BlockSpec((be, 8), lambda i: (i, 0)),
                  pl.BlockSpec((8, KP), lambda i: (0, 0))],
        out_specs=pl.BlockSpec((be, KP), lambda i: (i, 0)),
        out_shape=jax.ShapeDtypeStruct((E, KP), jnp.float32),
    )(P, M)


_sc_mesh = plsc.VectorSubcoreMesh(core_axis_name="c", subcore_axis_name="s",
                                  num_cores=NC, num_subcores=NS)


_SC_SCRATCH = [
    pltpu.VMEM((B,), jnp.int32), pltpu.VMEM((B,), jnp.int32),
    pltpu.VMEM((B,), jnp.int32), pltpu.VMEM((B,), jnp.int32),
    pltpu.VMEM((B, KP), jnp.float32), pltpu.VMEM((B, KP), jnp.float32),
    pltpu.VMEM((B, ROWP), jnp.float32), pltpu.VMEM((B, ROWP), jnp.float32),
    pltpu.VMEM((BW,), jnp.float32), pltpu.VMEM((BW,), jnp.float32),
    pltpu.VMEM((BW,), jnp.int32), pltpu.VMEM((BW,), jnp.int32),
    pltpu.VMEM_SHARED((FW,), jnp.float32),
    pltpu.SemaphoreType.DMA,
    pltpu.SemaphoreType.DMA,
    pltpu.SemaphoreType.DMA,
]


def _sc_body(h_hbm, src_hbm, dst_hbm, g_hbm, out_hbm,
             src0, src1, dst0, dst1, g0, g1, rows0, rows1,
             msg0, msg1, idx0, idx1,
             acc_sh, msem, gsem, ssem):
    c = lax.axis_index("c")
    s = lax.axis_index("s")
    wid = s * NC + c
    base = wid * EPW

    srcs, dsts, gs = [src0, src1], [dst0, dst1], [g0, g1]
    rows, msgs, idxs = [rows0, rows1], [msg0, msg1], [idx0, idx1]

    zeros16 = jnp.zeros((16,), jnp.float32)
    iota16 = lax.iota(jnp.int32, 16)

    def _zrow(i, carry):
        msg0[pl.ds(i * 16, 16)] = zeros16
        return carry

    lax.fori_loop(0, BW // 16, _zrow, 0)

    def _zblk(j, carry):
        pltpu.async_copy(msg0, acc_sh.at[pl.ds(s * ZW + j * BW, BW)], msem)
        return carry

    lax.fori_loop(0, ZW // BW, _zblk, 0)

    def _zdrain(j, carry):
        pltpu.make_async_copy(msg0, acc_sh.at[pl.ds(s * ZW, BW)], msem).wait()
        return carry

    lax.fori_loop(0, ZW // BW, _zdrain, 0)
    plsc.subcore_barrier()

    deg_marker = jnp.where(iota16 == 8, jnp.float32(1.0), jnp.float32(0.0))

    def meta_issue(blk, mb):
        e0 = base + blk * B
        pltpu.async_copy(src_hbm.at[pl.ds(e0, B)], srcs[mb], msem)
        pltpu.async_copy(dst_hbm.at[pl.ds(e0, B)], dsts[mb], msem)
        pltpu.async_copy(g_hbm.at[pl.ds(e0, B)], gs[mb], msem)

    def meta_wait(mb):
        pltpu.make_async_copy(src_hbm.at[pl.ds(0, B)], srcs[mb], msem).wait()
        pltpu.make_async_copy(dst_hbm.at[pl.ds(0, B)], dsts[mb], msem).wait()
        pltpu.make_async_copy(g_hbm.at[pl.ds(0, B)], gs[mb], msem).wait()

    def gather_issue(mb, rb):
        h = B // 2
        pltpu.async_copy(h_hbm.at[srcs[mb].at[pl.ds(0, h)]],
                         rows[rb].at[pl.ds(0, h)], gsem)
        pltpu.async_copy(h_hbm.at[srcs[mb].at[pl.ds(h, h)]],
                         rows[rb].at[pl.ds(h, h)], gsem)

    def gather_wait(rb):
        h = B // 2
        pltpu.make_async_copy(h_hbm.at[srcs[0].at[pl.ds(0, h)]],
                              rows[rb].at[pl.ds(0, h)], gsem).wait()
        pltpu.make_async_copy(h_hbm.at[srcs[0].at[pl.ds(h, h)]],
                              rows[rb].at[pl.ds(h, h)], gsem).wait()

    def scat_issue(pb):
        pltpu.async_copy(msgs[pb], acc_sh.at[idxs[pb]], ssem, add=True)

    def scat_wait(pb):
        pltpu.make_async_copy(msgs[pb], acc_sh.at[idxs[pb]], ssem).wait()

    def _compute(bi):
        dst_v, g_v, rows_v, msg_v, idx_v = (
            dsts[bi], gs[bi], rows[bi], msgs[bi], idxs[bi])
        for q in range(B // 16):
            dgrp = dst_v[pl.ds(q * 16, 16)]
            for l in range(16):
                dm = lax.broadcast(dgrp[l] * HP, (16,))
                mrow = (q * 16 + l) * HP
                for j in range(HP // 16):
                    idx_v[pl.ds(mrow + j * 16, 16)] = dm + (iota16 + j * 16)

        def _tree(ts):
            while len(ts) > 1:
                nxt = [ts[i] + ts[i + 1] for i in range(0, len(ts) - 1, 2)]
                if len(ts) % 2:
                    nxt.append(ts[-1])
                ts = nxt
            return ts[0]

        @plsc.parallel_loop(0, B, step=1, unroll=2)
        def _edge(b):
            g_row = g_v[b]
            gks = [lax.broadcast(g_row[k], (16,)) for k in range(K)]
            t0 = [rows_v[b, pl.ds(k * HP, 16)] * gks[k] for k in range(K)]
            t1 = [rows_v[b, pl.ds(k * HP + 16, 16)] * gks[k] for k in range(K)]
            t2 = [rows_v[b, pl.ds(k * HP + 32, 16)] * gks[k] for k in range(K)]
            msg_v[pl.ds(b * HP, 16)] = _tree(t0)
            msg_v[pl.ds(b * HP + 16, 16)] = _tree(t1)
            msg_v[pl.ds(b * HP + 32, 16)] = _tree(t2 + [deg_marker])

    def _sub(i, bi):
        @pl.when(i + 1 < NB)
        def _():
            meta_wait(1 - bi)
            gather_issue(1 - bi, 1 - bi)
        gather_wait(bi)
        @pl.when(i >= 2)
        def _():
            scat_wait(bi)
        _compute(bi)
        @pl.when(i + 2 < NB)
        def _():
            meta_issue(i + 2, bi)
        scat_issue(bi)

    meta_issue(0, 0)
    meta_wait(0)
    gather_issue(0, 0)
    meta_issue(1, 1)

    def _pair(i2, carry):
        i = i2 * 2
        _sub(i, 0)
        _sub(i + 1, 1)
        return carry

    lax.fori_loop(0, NB // 2, _pair, 0)

    scat_wait(0)
    scat_wait(1)
    plsc.subcore_barrier()

    pltpu.sync_copy(acc_sh.at[pl.ds(s * ZW, ZW)],
                    out_hbm.at[c, pl.ds(s * ZW, ZW)])


_sc_aggregate_impl = pl.kernel(
    _sc_body,
    out_type=jax.ShapeDtypeStruct((NC, FW), jnp.float32),
    mesh=_sc_mesh,
    scratch_types=_SC_SCRATCH,
)


def _sc_aggregate(h, src, dst, g):
    acc = _sc_aggregate_impl(h, src, dst, g)
    return acc.reshape(NC, NP, HP)[:, :N, :]


def _layer2_body(a0_ref, a1_ref, b_ref, w_ref, o_ref):
    s = a0_ref[...] + a1_ref[...]
    col = lax.broadcasted_iota(jnp.int32, s.shape, 1)
    deg = jnp.max(jnp.where(col == H, s, -jnp.inf), axis=1, keepdims=True)
    deg = jnp.maximum(deg, 1.0)
    hfeat = jnp.maximum(s / deg + b_ref[...], 0.0)
    o_ref[...] = jnp.dot(hfeat, w_ref[...],
                         preferred_element_type=jnp.float32)


def _final_body(a0_ref, a1_ref, b_ref, o_ref):
    s = a0_ref[...] + a1_ref[...]
    col = lax.broadcasted_iota(jnp.int32, s.shape, 1)
    deg = jnp.max(jnp.where(col == OUT, s, -jnp.inf), axis=1, keepdims=True)
    deg = jnp.maximum(deg, 1.0)
    o_ref[...] = jnp.tanh(s / deg + b_ref[...])


def kernel(n_feat, edge_index, p_coord, W1, mu1, inv_sigma1, b1,
           W2, mu2, inv_sigma2, b2):
    f32 = jnp.float32
    src = edge_index[0].astype(jnp.int32)
    dst = edge_index[1].astype(jnp.int32)
    p = p_coord.astype(f32)

    def padw(W, in_dim, out_dim):
        Wk = W.reshape(in_dim, K, out_dim)
        Wk = jnp.pad(Wk, ((0, 0), (0, 0), (0, HP - out_dim)))
        Wk = Wk.reshape(in_dim, K * HP)
        return jnp.pad(Wk, ((0, 0), (0, ROWP - ROW)))

    W1p = padw(W1, IN, H)
    W2p = jnp.pad(padw(W2, H, OUT), ((0, HP - H), (0, 0)))
    b1p = jnp.pad(b1, (0, HP - H)).reshape(1, HP)
    b2p = jnp.pad(b2, (0, HP - OUT)).reshape(1, HP)

    def build_M(mu, inv):
        iv2 = (inv * inv).astype(f32)
        rows = jnp.concatenate([
            (-0.5 * iv2).T,
            (iv2 * mu).T,
            (-0.5 * (iv2 * mu * mu).sum(axis=1))[None, :],
            jnp.zeros((3, K), f32),
        ], axis=0)
        return jnp.pad(rows, ((0, 0), (0, KP - K)))

    M1 = build_M(mu1, inv_sigma1)
    M2 = build_M(mu2, inv_sigma2)
    P = jnp.concatenate(
        [p * p, p, jnp.ones((E, 1), f32), jnp.zeros((E, 3), f32)], axis=1)

    g1 = _gmm_weights(P, M1)
    g2 = _gmm_weights(P, M2)
    h1p = _matmul(n_feat, W1p, 1000)

    pad = EP - E
    src_p = jnp.concatenate([src, jnp.zeros((pad,), jnp.int32)])
    dst_p = jnp.concatenate([dst, jnp.full((pad,), N, jnp.int32)])
    g1p = jnp.concatenate([g1, jnp.zeros((pad, KP), f32)])
    g2p = jnp.concatenate([g2, jnp.zeros((pad, KP), f32)])

    acc1 = _sc_aggregate(h1p, src_p, dst_p, g1p)

    bm = 1000
    h2p = pl.pallas_call(
        _layer2_body,
        grid=(N // bm,),
        in_specs=[pl.BlockSpec((bm, HP), lambda i: (i, 0)),
                  pl.BlockSpec((bm, HP), lambda i: (i, 0)),
                  pl.BlockSpec((1, HP), lambda i: (0, 0)),
                  pl.BlockSpec((HP, ROWP), lambda i: (0, 0))],
        out_specs=pl.BlockSpec((bm, ROWP), lambda i: (i, 0)),
        out_shape=jax.ShapeDtypeStruct((N, ROWP), f32),
    )(acc1[0], acc1[1], b1p, W2p)

    acc2 = _sc_aggregate(h2p, src_p, dst_p, g2p)

    out48 = pl.pallas_call(
        _final_body,
        grid=(N // bm,),
        in_specs=[pl.BlockSpec((bm, HP), lambda i: (i, 0)),
                  pl.BlockSpec((bm, HP), lambda i: (i, 0)),
                  pl.BlockSpec((1, HP), lambda i: (0, 0))],
        out_specs=pl.BlockSpec((bm, HP), lambda i: (i, 0)),
        out_shape=jax.ShapeDtypeStruct((N, HP), f32),
    )(acc2[0], acc2[1], b2p)

    return out48[:, :OUT]
